# baseline (speedup 1.0000x reference)
import functools

import jax
import jax.numpy as jnp
from jax import lax
from jax.experimental import pallas as pl
from jax.experimental.pallas import tpu as pltpu
from jax.experimental.pallas import tpu_sc as plsc

_VOCAB = 100000
_HIDDEN = 64
_BATCH = 4096
_VBLK = 512
_NSTEP = (_VOCAB + _VBLK - 1) // _VBLK
_TAIL = _VOCAB - (_NSTEP - 1) * _VBLK
_NBUF = 4


def _make_sc_gather():
    info = plsc.get_sparse_core_info()
    nc, ns = info.num_cores, info.num_subcores
    nw = nc * ns
    b_per_w = _BATCH // nw
    mesh = plsc.VectorSubcoreMesh(core_axis_name="c", subcore_axis_name="s")

    @functools.partial(
        pl.kernel,
        mesh=mesh,
        out_type=jax.ShapeDtypeStruct((_BATCH, _HIDDEN), jnp.float32),
        scratch_types=[
            pltpu.VMEM((b_per_w,), jnp.int32),
            pltpu.VMEM((b_per_w, _HIDDEN), jnp.float32),
            pltpu.SemaphoreType.DMA,
        ],
        compiler_params=pltpu.CompilerParams(use_tc_tiling_on_sc=False),
    )
    def gather_rows(idx_hbm, table_hbm, out_hbm, idx_v, rows_v, sem):
        wid = lax.axis_index("s") * nc + lax.axis_index("c")
        base = wid * b_per_w
        pltpu.sync_copy(idx_hbm.at[pl.ds(base, b_per_w)], idx_v)
        pltpu.async_copy(table_hbm.at[idx_v], rows_v, sem).wait()
        pltpu.sync_copy(rows_v, out_hbm.at[pl.ds(base, b_per_w)])

    return gather_rows


def _full_copy(buf, o_hbm, sems, jj, slot):
    return pltpu.make_async_copy(
        buf.at[slot],
        o_hbm.at[:, pl.ds(jj * _VBLK, _VBLK)],
        sems.at[slot],
    )


def _tail_copy(tbuf, o_hbm, sems, slot):
    return pltpu.make_async_copy(
        tbuf,
        o_hbm.at[:, pl.ds((_NSTEP - 1) * _VBLK, _TAIL)],
        sems.at[slot],
    )


def _mm_body(x_ref, w_ref, b_ref, o_hbm, buf, tbuf, sems):
    j = pl.program_id(0)
    slot = lax.rem(j, _NBUF)

    @pl.when(j >= _NBUF)
    def _():
        _full_copy(buf, o_hbm, sems, j - _NBUF, slot).wait()

    blk = (
        jnp.dot(x_ref[...], w_ref[...], preferred_element_type=jnp.float32)
        + b_ref[...]
    )

    @pl.when(j < _NSTEP - 1)
    def _():
        for k in range(_NBUF):
            @pl.when(slot == k)
            def _(k=k):
                buf[k] = blk
        _full_copy(buf, o_hbm, sems, j, slot).start()

    @pl.when(j == _NSTEP - 1)
    def _():
        tbuf[...] = blk[:, : _TAIL]
        _tail_copy(tbuf, o_hbm, sems, slot).start()
        for d in range(_NBUF - 1, 0, -1):
            jj = _NSTEP - 1 - d
            _full_copy(buf, o_hbm, sems, jj, jj % _NBUF).wait()
        _tail_copy(tbuf, o_hbm, sems, slot).wait()


def kernel(input_ids, embed_table, head_w, head_b):
    x = _make_sc_gather()(input_ids.astype(jnp.int32), embed_table)
    bias2d = head_b.reshape(1, _VOCAB)
    logits = pl.pallas_call(
        _mm_body,
        grid=(_NSTEP,),
        in_specs=[
            pl.BlockSpec((_BATCH, _HIDDEN), lambda j: (0, 0)),
            pl.BlockSpec((_HIDDEN, _VBLK), lambda j: (0, j)),
            pl.BlockSpec((1, _VBLK), lambda j: (0, j)),
        ],
        out_specs=pl.BlockSpec(memory_space=pltpu.MemorySpace.HBM),
        out_shape=jax.ShapeDtypeStruct((_BATCH, _VOCAB), jnp.float32),
        scratch_shapes=[
            pltpu.VMEM((_NBUF, _BATCH, _VBLK), jnp.float32),
            pltpu.VMEM((_BATCH, _TAIL), jnp.float32),
            pltpu.SemaphoreType.DMA((_NBUF,)),
        ],
        compiler_params=pltpu.CompilerParams(
            dimension_semantics=("arbitrary",)
        ),
    )(x, head_w, bias2d)
    return logits


# speedup vs baseline: 3.0723x; 3.0723x over previous
import functools

import jax
import jax.numpy as jnp
from jax import lax
from jax.experimental import pallas as pl
from jax.experimental.pallas import tpu as pltpu
from jax.experimental.pallas import tpu_sc as plsc

_VOCAB = 100000
_HIDDEN = 64
_BATCH = 4096
_VBLK = 512
_NSTEP = (_VOCAB + _VBLK - 1) // _VBLK
_TAIL = _VOCAB - (_NSTEP - 1) * _VBLK
_NBUF = 4


def _make_sc_gather():
    info = plsc.get_sparse_core_info()
    nc, ns = info.num_cores, info.num_subcores
    nw = nc * ns
    b_per_w = _BATCH // nw
    mesh = plsc.VectorSubcoreMesh(core_axis_name="c", subcore_axis_name="s")

    @functools.partial(
        pl.kernel,
        mesh=mesh,
        out_type=jax.ShapeDtypeStruct((_BATCH, _HIDDEN), jnp.float32),
        scratch_types=[
            pltpu.VMEM((b_per_w,), jnp.int32),
            pltpu.VMEM((b_per_w, _HIDDEN), jnp.float32),
            pltpu.SemaphoreType.DMA,
        ],
        compiler_params=pltpu.CompilerParams(use_tc_tiling_on_sc=False),
    )
    def gather_rows(idx_hbm, table_hbm, out_hbm, idx_v, rows_v, sem):
        wid = lax.axis_index("s") * nc + lax.axis_index("c")
        base = wid * b_per_w
        pltpu.sync_copy(idx_hbm.at[pl.ds(base, b_per_w)], idx_v)
        pltpu.async_copy(table_hbm.at[idx_v], rows_v, sem).wait()
        pltpu.sync_copy(rows_v, out_hbm.at[pl.ds(base, b_per_w)])

    return gather_rows


def _mm_body(wt_ref, xt_ref, b_ref, o_ref):
    o_ref[...] = (
        jnp.dot(wt_ref[...], xt_ref[...], preferred_element_type=jnp.float32)
        + b_ref[...]
    )


def kernel(input_ids, embed_table, head_w, head_b):
    x = _make_sc_gather()(input_ids.astype(jnp.int32), embed_table)
    xt = x.T
    wt = head_w.T
    bias2d = head_b.reshape(_VOCAB, 1)
    logits_t = pl.pallas_call(
        _mm_body,
        grid=(_NSTEP,),
        in_specs=[
            pl.BlockSpec((_VBLK, _HIDDEN), lambda j: (j, 0)),
            pl.BlockSpec((_HIDDEN, _BATCH), lambda j: (0, 0)),
            pl.BlockSpec((_VBLK, 1), lambda j: (j, 0)),
        ],
        out_specs=pl.BlockSpec((_VBLK, _BATCH), lambda j: (j, 0)),
        out_shape=jax.ShapeDtypeStruct((_VOCAB, _BATCH), jnp.float32),
        compiler_params=pltpu.CompilerParams(
            dimension_semantics=("arbitrary",)
        ),
    )(wt, xt, bias2d)
    return logits_t.T


# speedup vs baseline: 3.4160x; 1.1119x over previous
import functools

import jax
import jax.numpy as jnp
from jax import lax
from jax.experimental import pallas as pl
from jax.experimental.pallas import tpu as pltpu
from jax.experimental.pallas import tpu_sc as plsc

_VOCAB = 100000
_HIDDEN = 64
_BATCH = 4096
_VBLK = 512
_NSTEP = (_VOCAB + _VBLK - 1) // _VBLK
_TAIL = _VOCAB - (_NSTEP - 1) * _VBLK
_NBUF = 4


def _make_sc_gather():
    info = plsc.get_sparse_core_info()
    nc, ns = info.num_cores, info.num_subcores
    nw = nc * ns
    b_per_w = _BATCH // nw
    mesh = plsc.VectorSubcoreMesh(core_axis_name="c", subcore_axis_name="s")

    @functools.partial(
        pl.kernel,
        mesh=mesh,
        out_type=jax.ShapeDtypeStruct((_BATCH, _HIDDEN), jnp.float32),
        scratch_types=[
            pltpu.VMEM((b_per_w,), jnp.int32),
            pltpu.VMEM((b_per_w, _HIDDEN), jnp.float32),
            pltpu.SemaphoreType.DMA,
        ],
        compiler_params=pltpu.CompilerParams(use_tc_tiling_on_sc=False),
    )
    def gather_rows(idx_hbm, table_hbm, out_hbm, idx_v, rows_v, sem):
        wid = lax.axis_index("s") * nc + lax.axis_index("c")
        base = wid * b_per_w
        pltpu.sync_copy(idx_hbm.at[pl.ds(base, b_per_w)], idx_v)
        pltpu.async_copy(table_hbm.at[idx_v], rows_v, sem).wait()
        pltpu.sync_copy(rows_v, out_hbm.at[pl.ds(base, b_per_w)])

    return gather_rows


def _mm_body(w_ref, xt_ref, b_ref, o_ref):
    prod = lax.dot_general(
        w_ref[...],
        xt_ref[...],
        (((0,), (0,)), ((), ())),
        preferred_element_type=jnp.float32,
    )
    o_ref[...] = prod + jnp.transpose(b_ref[...], (1, 0))


def kernel(input_ids, embed_table, head_w, head_b):
    x = _make_sc_gather()(input_ids.astype(jnp.int32), embed_table)
    xt = x.T
    bias2d = head_b.reshape(1, _VOCAB)
    logits_t = pl.pallas_call(
        _mm_body,
        grid=(_NSTEP,),
        in_specs=[
            pl.BlockSpec((_HIDDEN, _VBLK), lambda j: (0, j)),
            pl.BlockSpec((_HIDDEN, _BATCH), lambda j: (0, 0)),
            pl.BlockSpec((1, _VBLK), lambda j: (0, j)),
        ],
        out_specs=pl.BlockSpec((_VBLK, _BATCH), lambda j: (j, 0)),
        out_shape=jax.ShapeDtypeStruct((_VOCAB, _BATCH), jnp.float32),
        compiler_params=pltpu.CompilerParams(
            dimension_semantics=("arbitrary",)
        ),
    )(head_w, xt, bias2d)
    return logits_t.T


# speedup vs baseline: 3.4569x; 1.0120x over previous
import functools

import jax
import jax.numpy as jnp
from jax import lax
from jax.experimental import pallas as pl
from jax.experimental.pallas import tpu as pltpu
from jax.experimental.pallas import tpu_sc as plsc

_VOCAB = 100000
_HIDDEN = 64
_HPAD = 128
_BATCH = 4096
_VBLK = 512
_NSTEP = (_VOCAB + _VBLK - 1) // _VBLK


def _make_sc_gather():
    info = plsc.get_sparse_core_info()
    nc, ns = info.num_cores, info.num_subcores
    nw = nc * ns
    b_per_w = _BATCH // nw
    mesh = plsc.VectorSubcoreMesh(core_axis_name="c", subcore_axis_name="s")

    @functools.partial(
        pl.kernel,
        mesh=mesh,
        out_type=jax.ShapeDtypeStruct((_BATCH, _HPAD), jnp.float32),
        scratch_types=[
            pltpu.VMEM((b_per_w,), jnp.int32),
            pltpu.VMEM((b_per_w, _HPAD), jnp.float32),
            pltpu.SemaphoreType.DMA,
        ],
    )
    def gather_rows(idx_hbm, table_hbm, out_hbm, idx_v, rows_v, sem):
        wid = lax.axis_index("s") * nc + lax.axis_index("c")
        base = wid * b_per_w
        pltpu.sync_copy(idx_hbm.at[pl.ds(base, b_per_w)], idx_v)
        pltpu.async_copy(table_hbm.at[idx_v], rows_v, sem).wait()
        pltpu.sync_copy(rows_v, out_hbm.at[pl.ds(base, b_per_w)])

    return gather_rows


def _mm_body(w_ref, xt_ref, b_ref, o_ref):
    prod = lax.dot_general(
        w_ref[...],
        xt_ref[...],
        (((0,), (0,)), ((), ())),
        preferred_element_type=jnp.float32,
    )
    o_ref[...] = prod + jnp.transpose(b_ref[...], (1, 0))


def kernel(input_ids, embed_table, head_w, head_b):
    table_pad = jnp.pad(embed_table, ((0, 0), (0, _HPAD - _HIDDEN)))
    x_pad = _make_sc_gather()(input_ids.astype(jnp.int32), table_pad)
    xt = x_pad.T[: _HIDDEN]
    bias2d = head_b.reshape(1, _VOCAB)
    logits_t = pl.pallas_call(
        _mm_body,
        grid=(_NSTEP,),
        in_specs=[
            pl.BlockSpec((_HIDDEN, _VBLK), lambda j: (0, j)),
            pl.BlockSpec((_HIDDEN, _BATCH), lambda j: (0, 0)),
            pl.BlockSpec((1, _VBLK), lambda j: (0, j)),
        ],
        out_specs=pl.BlockSpec((_VBLK, _BATCH), lambda j: (j, 0)),
        out_shape=jax.ShapeDtypeStruct((_VOCAB, _BATCH), jnp.float32),
        compiler_params=pltpu.CompilerParams(
            dimension_semantics=("arbitrary",)
        ),
    )(head_w, xt, bias2d)
    return logits_t.T
